# baseline (speedup 1.0000x reference)
import functools

import jax
import jax.numpy as jnp
from jax import lax
from jax.experimental import pallas as pl
from jax.experimental.pallas import tpu as pltpu
from jax.experimental.pallas import tpu_sc as plsc

_N = 10000
_E = 320000
_D = 128
_DEPTH = 3
_K = 96
_C = 105
_NC = 2
_NS = 16
_NW = _NC * _NS
_EPW = _C * _K
_EPAD = _NW * _EPW
_NPAD = 10240
_RPT = _NPAD // _NS
_BLK = 2000
_GRID = _N // _BLK


def _sc_scatter_rows(table, src1, dst1, zeros, gather):
    w_dim = table.shape[1]
    mesh = plsc.VectorSubcoreMesh(core_axis_name="c", subcore_axis_name="s")

    @functools.partial(
        pl.kernel,
        out_type=jax.ShapeDtypeStruct((_NC, _NPAD, w_dim), jnp.float32),
        mesh=mesh,
        scratch_types=(
            [pltpu.VMEM((_EPW,), jnp.int32)]
            + [pltpu.VMEM((_K,), jnp.int32)] * _DEPTH
            + [pltpu.VMEM((_K, w_dim), jnp.float32)] * _DEPTH
            + [pltpu.SemaphoreType.DMA] * (2 * _DEPTH)
            + [pltpu.VMEM_SHARED((_NPAD, w_dim), jnp.float32)]
        ),
    )
    def k(table_hbm, src_hbm, dst_hbm, zeros_hbm, out_hbm, srcv, *rest):
        dbufs = rest[:_DEPTH]
        rbufs = rest[_DEPTH:2 * _DEPTH]
        gsems = rest[2 * _DEPTH:3 * _DEPTH]
        dsems = rest[3 * _DEPTH:4 * _DEPTH]
        accum = rest[-1]
        c = lax.axis_index("c")
        s = lax.axis_index("s")
        w = c * _NS + s
        base = w * _EPW

        pltpu.sync_copy(zeros_hbm, accum.at[pl.ds(s * _RPT, _RPT)])
        if gather:
            pltpu.sync_copy(src_hbm.at[pl.ds(base, _EPW)], srcv)
        else:
            pltpu.sync_copy(table_hbm, rbufs[0])

        def issue_d(j, b):
            pltpu.async_copy(dst_hbm.at[pl.ds(base + j * _K, _K)],
                             dbufs[b], dsems[b])

        def wait_d(j, b):
            pltpu.make_async_copy(dst_hbm.at[pl.ds(base + j * _K, _K)],
                                  dbufs[b], dsems[b]).wait()

        def issue_g(j, b):
            pltpu.async_copy(table_hbm.at[srcv.at[pl.ds(j * _K, _K)]],
                             rbufs[b], gsems[b])

        def wait_g(j, b):
            pltpu.make_async_copy(table_hbm.at[srcv.at[pl.ds(j * _K, _K)]],
                                  rbufs[b], gsems[b]).wait()

        for b in range(_DEPTH):
            issue_d(b, b)
            if gather:
                issue_g(b, b)
        plsc.subcore_barrier()

        if gather:
            def body(i, carry):
                j0 = _DEPTH * i
                for b in range(_DEPTH):
                    wait_g(j0 + b, b)
                    wait_d(j0 + b, b)
                    pltpu.sync_copy(rbufs[b], accum.at[dbufs[b]], add=True)
                    issue_g(j0 + b + _DEPTH, b)
                    issue_d(j0 + b + _DEPTH, b)
                return carry
        else:
            def body(i, carry):
                j0 = _DEPTH * i
                for b in range(_DEPTH):
                    wait_d(j0 + b, b)
                    pltpu.sync_copy(rbufs[0], accum.at[dbufs[b]], add=True)
                    issue_d(j0 + b + _DEPTH, b)
                return carry

        lax.fori_loop(0, _C // _DEPTH - 1, body, 0)
        for b in range(_DEPTH):
            j = _C - _DEPTH + b
            if gather:
                wait_g(j, b)
                wait_d(j, b)
                pltpu.sync_copy(rbufs[b], accum.at[dbufs[b]], add=True)
            else:
                wait_d(j, b)
                pltpu.sync_copy(rbufs[0], accum.at[dbufs[b]], add=True)
        plsc.subcore_barrier()
        pltpu.sync_copy(accum.at[pl.ds(s * _RPT, _RPT)],
                        out_hbm.at[c, pl.ds(s * _RPT, _RPT)])

    return k(table, src1, dst1, zeros)


def _dot(a, b):
    return lax.dot_general(a, b, (((1,), (0,)), ((), ())),
                           preferred_element_type=jnp.float32)


def _ln(t, g, b):
    m = jnp.mean(t, axis=-1, keepdims=True)
    v = jnp.mean((t - m) ** 2, axis=-1, keepdims=True)
    return (t - m) * lax.rsqrt(v + 1e-5) * g + b


def _gelu(t):
    return 0.5 * t * (1.0 + lax.erf(t * 0.7071067811865476))


def _dinv_of(dp_ref):
    deg = dp_ref[0] + dp_ref[1] + 1.0
    return lax.rsqrt(deg)


_ROW = lambda i: (i, 0)
_FIX = lambda i: (0, 0)
_P0 = lambda i: (0, i, 0)
_P1 = lambda i: (1, i, 0)


def _spec(shape, imap):
    return pl.BlockSpec(shape, imap)


def _pair_spec():
    return pl.BlockSpec((2, _BLK, _D), lambda i: (0, i, 0))


def _deg_spec():
    return pl.BlockSpec((2, _BLK, 1), lambda i: (0, i, 0))


def _tc0a_body(x_ref, pw_ref, pb_ref, g1_ref, b1_ref, h0_ref):
    h = _dot(x_ref[...], pw_ref[...]) + pb_ref[...]
    h0_ref[...] = _ln(h, g1_ref[...], b1_ref[...])


def _tc_proj_a(x, proj_W, proj_b, ln1_g, ln1_b):
    return pl.pallas_call(
        _tc0a_body,
        grid=(_GRID,),
        in_specs=[
            _spec((_BLK, _D), _ROW),
            _spec((_D, _D), _FIX),
            _spec((1, _D), _FIX),
            _spec((1, _D), _FIX),
            _spec((1, _D), _FIX),
        ],
        out_specs=_spec((_BLK, _D), _ROW),
        out_shape=jax.ShapeDtypeStruct((_N, _D), jnp.float32),
    )(x, proj_W, proj_b, ln1_g, ln1_b)


def _tc0b_body(h0_ref, dp_ref, w0_ref, gs0_ref):
    dinv = _dinv_of(dp_ref)
    gs0_ref[...] = _dot(h0_ref[...], w0_ref[...]) * dinv


def _tc_proj_b(h0, degp, W0):
    return pl.pallas_call(
        _tc0b_body,
        grid=(_GRID,),
        in_specs=[
            _spec((_BLK, _D), _ROW),
            _deg_spec(),
            _spec((_D, _D), _FIX),
        ],
        out_specs=_spec((_BLK, _D), _ROW),
        out_shape=jax.ShapeDtypeStruct((_N, _D), jnp.float32),
    )(h0, degp, W0)


def _hop_body(p_ref, gsp_ref, dp_ref, b_ref, wn_ref, h_ref, gs_ref):
    dinv = _dinv_of(dp_ref)
    agg = p_ref[0] + p_ref[1] + gsp_ref[...]
    h = _gelu(agg * dinv + b_ref[...])
    h_ref[...] = h
    gs_ref[...] = _dot(h, wn_ref[...]) * dinv


def _tc_hop(p, gsp, degp, bias, Wn):
    return pl.pallas_call(
        _hop_body,
        grid=(_GRID,),
        in_specs=[
            _pair_spec(),
            _spec((_BLK, _D), _ROW),
            _deg_spec(),
            _spec((1, _D), _FIX),
            _spec((_D, _D), _FIX),
        ],
        out_specs=[_spec((_BLK, _D), _ROW)] * 2,
        out_shape=[jax.ShapeDtypeStruct((_N, _D), jnp.float32)] * 2,
    )(p, gsp, degp, bias, Wn)


def _fin_body(p_ref, gs2_ref, dp_ref, b2_ref, h0_ref, h1_ref, h2_ref,
              w1_ref, b1m_ref, w2_ref, b2m_ref, g2_ref, bb2_ref, out_ref):
    dinv = _dinv_of(dp_ref)
    agg = p_ref[0] + p_ref[1] + gs2_ref[...]
    h3 = _gelu(agg * dinv + b2_ref[...])
    w1 = w1_ref[...]
    m = (_dot(h0_ref[...], w1[0:_D]) + _dot(h1_ref[...], w1[_D:2 * _D])
         + _dot(h2_ref[...], w1[2 * _D:3 * _D]) + _dot(h3, w1[3 * _D:4 * _D])
         + b1m_ref[...])
    m = _gelu(m)
    o = _dot(m, w2_ref[...]) + b2m_ref[...]
    out_ref[...] = _ln(o, g2_ref[...], bb2_ref[...])


def _tc_final(p, gs2, degp, b2, h0, h1, h2,
              mlp_W1, mlp_b1, mlp_W2, mlp_b2, ln2_g, ln2_b):
    return pl.pallas_call(
        _fin_body,
        grid=(_GRID,),
        in_specs=[
            _pair_spec(),
            _spec((_BLK, _D), _ROW),
            _deg_spec(),
            _spec((1, _D), _FIX),
            _spec((_BLK, _D), _ROW),
            _spec((_BLK, _D), _ROW),
            _spec((_BLK, _D), _ROW),
            _spec((4 * _D, _D), _FIX),
            _spec((1, _D), _FIX),
            _spec((_D, _D), _FIX),
            _spec((1, _D), _FIX),
            _spec((1, _D), _FIX),
            _spec((1, _D), _FIX),
        ],
        out_specs=_spec((_BLK, _D), _ROW),
        out_shape=jax.ShapeDtypeStruct((_N, _D), jnp.float32),
    )(p, gs2, degp, b2, h0, h1, h2,
      mlp_W1, mlp_b1, mlp_W2, mlp_b2, ln2_g, ln2_b)


def kernel(x, edge_index, proj_W, proj_b, ln1_g, ln1_b, gcn_Ws, gcn_bs,
           mlp_W1, mlp_b1, mlp_W2, mlp_b2, ln2_g, ln2_b):
    f32 = jnp.float32
    padw = _EPW - _E // _NW
    pad_src = jnp.zeros((_NW, padw), edge_index.dtype)
    pad_dst = jnp.broadcast_to(
        _N + jnp.arange(padw, dtype=edge_index.dtype), (_NW, padw))
    src1 = jnp.concatenate(
        [edge_index[0].reshape(_NW, _E // _NW), pad_src], axis=1).reshape(_EPAD)
    dst1 = jnp.concatenate(
        [edge_index[1].reshape(_NW, _E // _NW), pad_dst], axis=1).reshape(_EPAD)
    ones = jnp.ones((_K, _D), f32)
    zeros = jnp.zeros((_RPT, _D), f32)
    row = lambda v: v.reshape(1, _D)

    degp = _sc_scatter_rows(ones, src1, dst1, zeros, gather=False)[:, :_N, :1]

    h0 = _tc_proj_a(x, proj_W, row(proj_b), row(ln1_g), row(ln1_b))
    gs0 = _tc_proj_b(h0, degp, gcn_Ws[0])
    p = _sc_scatter_rows(gs0, src1, dst1, zeros, gather=True)
    h1, gs1 = _tc_hop(p, gs0, degp, row(gcn_bs[0]), gcn_Ws[1])
    p = _sc_scatter_rows(gs1, src1, dst1, zeros, gather=True)
    h2, gs2 = _tc_hop(p, gs1, degp, row(gcn_bs[1]), gcn_Ws[2])
    p = _sc_scatter_rows(gs2, src1, dst1, zeros, gather=True)
    return _tc_final(p, gs2, degp, row(gcn_bs[2]), h0, h1, h2,
                     mlp_W1, row(mlp_b1), mlp_W2, row(mlp_b2),
                     row(ln2_g), row(ln2_b))


# speedup vs baseline: 1.0002x; 1.0002x over previous
import functools

import jax
import jax.numpy as jnp
from jax import lax
from jax.experimental import pallas as pl
from jax.experimental.pallas import tpu as pltpu
from jax.experimental.pallas import tpu_sc as plsc

_N = 10000
_E = 320000
_D = 128
_DEPTH = 3
_K = 96
_C = 105
_NC = 2
_NS = 16
_NW = _NC * _NS
_EPW = _C * _K
_EPAD = _NW * _EPW
_NPAD = 10240
_RPT = _NPAD // _NS
_BLK = 2000
_GRID = _N // _BLK


def _sc_scatter_rows(table, src1, dst1, zeros, gather):
    w_dim = table.shape[1]
    mesh = plsc.VectorSubcoreMesh(core_axis_name="c", subcore_axis_name="s")

    @functools.partial(
        pl.kernel,
        out_type=jax.ShapeDtypeStruct((_NC, _NPAD, w_dim), jnp.float32),
        mesh=mesh,
        scratch_types=(
            [pltpu.VMEM((_EPW,), jnp.int32)]
            + [pltpu.VMEM((_K,), jnp.int32)] * _DEPTH
            + [pltpu.VMEM((_K, w_dim), jnp.float32)] * _DEPTH
            + [pltpu.SemaphoreType.DMA] * (2 * _DEPTH)
            + [pltpu.VMEM_SHARED((_NPAD, w_dim), jnp.float32)]
        ),
    )
    def k(table_hbm, src_hbm, dst_hbm, zeros_hbm, out_hbm, srcv, *rest):
        dbufs = rest[:_DEPTH]
        rbufs = rest[_DEPTH:2 * _DEPTH]
        gsems = rest[2 * _DEPTH:3 * _DEPTH]
        dsems = rest[3 * _DEPTH:4 * _DEPTH]
        accum = rest[-1]
        c = lax.axis_index("c")
        s = lax.axis_index("s")
        w = c * _NS + s
        base = w * _EPW

        pltpu.sync_copy(zeros_hbm, accum.at[pl.ds(s * _RPT, _RPT)])
        if gather:
            pltpu.sync_copy(src_hbm.at[pl.ds(base, _EPW)], srcv)
        else:
            pltpu.sync_copy(table_hbm, rbufs[0])

        def issue_d(j, b):
            pltpu.async_copy(dst_hbm.at[pl.ds(base + j * _K, _K)],
                             dbufs[b], dsems[b])

        def wait_d(j, b):
            pltpu.make_async_copy(dst_hbm.at[pl.ds(base + j * _K, _K)],
                                  dbufs[b], dsems[b]).wait()

        def issue_g(j, b):
            pltpu.async_copy(table_hbm.at[srcv.at[pl.ds(j * _K, _K)]],
                             rbufs[b], gsems[b], priority=1)

        def wait_g(j, b):
            pltpu.make_async_copy(table_hbm.at[srcv.at[pl.ds(j * _K, _K)]],
                                  rbufs[b], gsems[b]).wait()

        for b in range(_DEPTH):
            issue_d(b, b)
            if gather:
                issue_g(b, b)
        plsc.subcore_barrier()

        if gather:
            def body(i, carry):
                j0 = _DEPTH * i
                for b in range(_DEPTH):
                    wait_g(j0 + b, b)
                    wait_d(j0 + b, b)
                    pltpu.sync_copy(rbufs[b], accum.at[dbufs[b]], add=True)
                    issue_g(j0 + b + _DEPTH, b)
                    issue_d(j0 + b + _DEPTH, b)
                return carry
        else:
            def body(i, carry):
                j0 = _DEPTH * i
                for b in range(_DEPTH):
                    wait_d(j0 + b, b)
                    pltpu.sync_copy(rbufs[0], accum.at[dbufs[b]], add=True)
                    issue_d(j0 + b + _DEPTH, b)
                return carry

        lax.fori_loop(0, _C // _DEPTH - 1, body, 0)
        for b in range(_DEPTH):
            j = _C - _DEPTH + b
            if gather:
                wait_g(j, b)
                wait_d(j, b)
                pltpu.sync_copy(rbufs[b], accum.at[dbufs[b]], add=True)
            else:
                wait_d(j, b)
                pltpu.sync_copy(rbufs[0], accum.at[dbufs[b]], add=True)
        plsc.subcore_barrier()
        pltpu.sync_copy(accum.at[pl.ds(s * _RPT, _RPT)],
                        out_hbm.at[c, pl.ds(s * _RPT, _RPT)])

    return k(table, src1, dst1, zeros)


def _dot(a, b):
    return lax.dot_general(a, b, (((1,), (0,)), ((), ())),
                           preferred_element_type=jnp.float32)


def _ln(t, g, b):
    m = jnp.mean(t, axis=-1, keepdims=True)
    v = jnp.mean((t - m) ** 2, axis=-1, keepdims=True)
    return (t - m) * lax.rsqrt(v + 1e-5) * g + b


def _gelu(t):
    return 0.5 * t * (1.0 + lax.erf(t * 0.7071067811865476))


def _dinv_of(dp_ref):
    deg = dp_ref[0] + dp_ref[1] + 1.0
    return lax.rsqrt(deg)


_ROW = lambda i: (i, 0)
_FIX = lambda i: (0, 0)
_P0 = lambda i: (0, i, 0)
_P1 = lambda i: (1, i, 0)


def _spec(shape, imap):
    return pl.BlockSpec(shape, imap)


def _pair_spec():
    return pl.BlockSpec((2, _BLK, _D), lambda i: (0, i, 0))


def _deg_spec():
    return pl.BlockSpec((2, _BLK, 1), lambda i: (0, i, 0))


def _tc0a_body(x_ref, pw_ref, pb_ref, g1_ref, b1_ref, h0_ref):
    h = _dot(x_ref[...], pw_ref[...]) + pb_ref[...]
    h0_ref[...] = _ln(h, g1_ref[...], b1_ref[...])


def _tc_proj_a(x, proj_W, proj_b, ln1_g, ln1_b):
    return pl.pallas_call(
        _tc0a_body,
        grid=(_GRID,),
        in_specs=[
            _spec((_BLK, _D), _ROW),
            _spec((_D, _D), _FIX),
            _spec((1, _D), _FIX),
            _spec((1, _D), _FIX),
            _spec((1, _D), _FIX),
        ],
        out_specs=_spec((_BLK, _D), _ROW),
        out_shape=jax.ShapeDtypeStruct((_N, _D), jnp.float32),
    )(x, proj_W, proj_b, ln1_g, ln1_b)


def _tc0b_body(h0_ref, dp_ref, w0_ref, gs0_ref):
    dinv = _dinv_of(dp_ref)
    gs0_ref[...] = _dot(h0_ref[...], w0_ref[...]) * dinv


def _tc_proj_b(h0, degp, W0):
    return pl.pallas_call(
        _tc0b_body,
        grid=(_GRID,),
        in_specs=[
            _spec((_BLK, _D), _ROW),
            _deg_spec(),
            _spec((_D, _D), _FIX),
        ],
        out_specs=_spec((_BLK, _D), _ROW),
        out_shape=jax.ShapeDtypeStruct((_N, _D), jnp.float32),
    )(h0, degp, W0)


def _hop_body(p_ref, gsp_ref, dp_ref, b_ref, wn_ref, h_ref, gs_ref):
    dinv = _dinv_of(dp_ref)
    agg = p_ref[0] + p_ref[1] + gsp_ref[...]
    h = _gelu(agg * dinv + b_ref[...])
    h_ref[...] = h
    gs_ref[...] = _dot(h, wn_ref[...]) * dinv


def _tc_hop(p, gsp, degp, bias, Wn):
    return pl.pallas_call(
        _hop_body,
        grid=(_GRID,),
        in_specs=[
            _pair_spec(),
            _spec((_BLK, _D), _ROW),
            _deg_spec(),
            _spec((1, _D), _FIX),
            _spec((_D, _D), _FIX),
        ],
        out_specs=[_spec((_BLK, _D), _ROW)] * 2,
        out_shape=[jax.ShapeDtypeStruct((_N, _D), jnp.float32)] * 2,
    )(p, gsp, degp, bias, Wn)


def _fin_body(p_ref, gs2_ref, dp_ref, b2_ref, h0_ref, h1_ref, h2_ref,
              w1_ref, b1m_ref, w2_ref, b2m_ref, g2_ref, bb2_ref, out_ref):
    dinv = _dinv_of(dp_ref)
    agg = p_ref[0] + p_ref[1] + gs2_ref[...]
    h3 = _gelu(agg * dinv + b2_ref[...])
    w1 = w1_ref[...]
    m = (_dot(h0_ref[...], w1[0:_D]) + _dot(h1_ref[...], w1[_D:2 * _D])
         + _dot(h2_ref[...], w1[2 * _D:3 * _D]) + _dot(h3, w1[3 * _D:4 * _D])
         + b1m_ref[...])
    m = _gelu(m)
    o = _dot(m, w2_ref[...]) + b2m_ref[...]
    out_ref[...] = _ln(o, g2_ref[...], bb2_ref[...])


def _tc_final(p, gs2, degp, b2, h0, h1, h2,
              mlp_W1, mlp_b1, mlp_W2, mlp_b2, ln2_g, ln2_b):
    return pl.pallas_call(
        _fin_body,
        grid=(_GRID,),
        in_specs=[
            _pair_spec(),
            _spec((_BLK, _D), _ROW),
            _deg_spec(),
            _spec((1, _D), _FIX),
            _spec((_BLK, _D), _ROW),
            _spec((_BLK, _D), _ROW),
            _spec((_BLK, _D), _ROW),
            _spec((4 * _D, _D), _FIX),
            _spec((1, _D), _FIX),
            _spec((_D, _D), _FIX),
            _spec((1, _D), _FIX),
            _spec((1, _D), _FIX),
            _spec((1, _D), _FIX),
        ],
        out_specs=_spec((_BLK, _D), _ROW),
        out_shape=jax.ShapeDtypeStruct((_N, _D), jnp.float32),
    )(p, gs2, degp, b2, h0, h1, h2,
      mlp_W1, mlp_b1, mlp_W2, mlp_b2, ln2_g, ln2_b)


def kernel(x, edge_index, proj_W, proj_b, ln1_g, ln1_b, gcn_Ws, gcn_bs,
           mlp_W1, mlp_b1, mlp_W2, mlp_b2, ln2_g, ln2_b):
    f32 = jnp.float32
    padw = _EPW - _E // _NW
    pad_src = jnp.zeros((_NW, padw), edge_index.dtype)
    pad_dst = jnp.broadcast_to(
        _N + jnp.arange(padw, dtype=edge_index.dtype), (_NW, padw))
    src1 = jnp.concatenate(
        [edge_index[0].reshape(_NW, _E // _NW), pad_src], axis=1).reshape(_EPAD)
    dst1 = jnp.concatenate(
        [edge_index[1].reshape(_NW, _E // _NW), pad_dst], axis=1).reshape(_EPAD)
    ones = jnp.ones((_K, _D), f32)
    zeros = jnp.zeros((_RPT, _D), f32)
    row = lambda v: v.reshape(1, _D)

    degp = _sc_scatter_rows(ones, src1, dst1, zeros, gather=False)[:, :_N, :1]

    h0 = _tc_proj_a(x, proj_W, row(proj_b), row(ln1_g), row(ln1_b))
    gs0 = _tc_proj_b(h0, degp, gcn_Ws[0])
    p = _sc_scatter_rows(gs0, src1, dst1, zeros, gather=True)
    h1, gs1 = _tc_hop(p, gs0, degp, row(gcn_bs[0]), gcn_Ws[1])
    p = _sc_scatter_rows(gs1, src1, dst1, zeros, gather=True)
    h2, gs2 = _tc_hop(p, gs1, degp, row(gcn_bs[1]), gcn_Ws[2])
    p = _sc_scatter_rows(gs2, src1, dst1, zeros, gather=True)
    return _tc_final(p, gs2, degp, row(gcn_bs[2]), h0, h1, h2,
                     mlp_W1, row(mlp_b1), mlp_W2, row(mlp_b2),
                     row(ln2_g), row(ln2_b))


# speedup vs baseline: 1.0084x; 1.0082x over previous
import functools

import jax
import jax.numpy as jnp
from jax import lax
from jax.experimental import pallas as pl
from jax.experimental.pallas import tpu as pltpu
from jax.experimental.pallas import tpu_sc as plsc

_N = 10000
_E = 320000
_D = 128
_DEPTH = 3
_K = 96
_C = 105
_NC = 2
_NS = 16
_NW = _NC * _NS
_EPW = _C * _K
_EPAD = _NW * _EPW
_NPAD = 10112
_RPT = _NPAD // _NS
_BLK = 2000
_GRID = _N // _BLK


def _sc_scatter_rows(table, src1, dst1, zeros, gather):
    w_dim = table.shape[1]
    mesh = plsc.VectorSubcoreMesh(core_axis_name="c", subcore_axis_name="s")

    @functools.partial(
        pl.kernel,
        out_type=jax.ShapeDtypeStruct((_NC, _NPAD, w_dim), jnp.float32),
        mesh=mesh,
        scratch_types=(
            [pltpu.VMEM((_EPW,), jnp.int32)]
            + [pltpu.VMEM((_K,), jnp.int32)] * _DEPTH
            + [pltpu.VMEM((_K, w_dim), jnp.float32)] * _DEPTH
            + [pltpu.SemaphoreType.DMA] * (2 * _DEPTH)
            + [pltpu.VMEM_SHARED((_NPAD, w_dim), jnp.float32)]
        ),
    )
    def k(table_hbm, src_hbm, dst_hbm, zeros_hbm, out_hbm, srcv, *rest):
        dbufs = rest[:_DEPTH]
        rbufs = rest[_DEPTH:2 * _DEPTH]
        gsems = rest[2 * _DEPTH:3 * _DEPTH]
        dsems = rest[3 * _DEPTH:4 * _DEPTH]
        accum = rest[-1]
        c = lax.axis_index("c")
        s = lax.axis_index("s")
        w = c * _NS + s
        base = w * _EPW

        pltpu.sync_copy(zeros_hbm, accum.at[pl.ds(s * _RPT, _RPT)])
        if gather:
            pltpu.sync_copy(src_hbm.at[pl.ds(base, _EPW)], srcv)
        else:
            pltpu.sync_copy(table_hbm, rbufs[0])

        def issue_d(j, b):
            pltpu.async_copy(dst_hbm.at[pl.ds(base + j * _K, _K)],
                             dbufs[b], dsems[b])

        def wait_d(j, b):
            pltpu.make_async_copy(dst_hbm.at[pl.ds(base + j * _K, _K)],
                                  dbufs[b], dsems[b]).wait()

        def issue_g(j, b):
            pltpu.async_copy(table_hbm.at[srcv.at[pl.ds(j * _K, _K)]],
                             rbufs[b], gsems[b])

        def wait_g(j, b):
            pltpu.make_async_copy(table_hbm.at[srcv.at[pl.ds(j * _K, _K)]],
                                  rbufs[b], gsems[b]).wait()

        for b in range(_DEPTH):
            issue_d(b, b)
            if gather:
                issue_g(b, b)
        plsc.subcore_barrier()

        if gather:
            def body(i, carry):
                j0 = _DEPTH * i
                for b in range(_DEPTH):
                    wait_g(j0 + b, b)
                    wait_d(j0 + b, b)
                    pltpu.sync_copy(rbufs[b], accum.at[dbufs[b]], add=True)
                    issue_g(j0 + b + _DEPTH, b)
                    issue_d(j0 + b + _DEPTH, b)
                return carry
        else:
            def body(i, carry):
                j0 = _DEPTH * i
                for b in range(_DEPTH):
                    wait_d(j0 + b, b)
                    pltpu.sync_copy(rbufs[0], accum.at[dbufs[b]], add=True)
                    issue_d(j0 + b + _DEPTH, b)
                return carry

        lax.fori_loop(0, _C // _DEPTH - 1, body, 0)
        for b in range(_DEPTH):
            j = _C - _DEPTH + b
            if gather:
                wait_g(j, b)
                wait_d(j, b)
                pltpu.sync_copy(rbufs[b], accum.at[dbufs[b]], add=True)
            else:
                wait_d(j, b)
                pltpu.sync_copy(rbufs[0], accum.at[dbufs[b]], add=True)
        plsc.subcore_barrier()
        pltpu.sync_copy(accum.at[pl.ds(s * _RPT, _RPT)],
                        out_hbm.at[c, pl.ds(s * _RPT, _RPT)])

    return k(table, src1, dst1, zeros)


def _dot(a, b):
    return lax.dot_general(a, b, (((1,), (0,)), ((), ())),
                           preferred_element_type=jnp.float32)


def _ln(t, g, b):
    m = jnp.mean(t, axis=-1, keepdims=True)
    v = jnp.mean((t - m) ** 2, axis=-1, keepdims=True)
    return (t - m) * lax.rsqrt(v + 1e-5) * g + b


def _gelu(t):
    return 0.5 * t * (1.0 + lax.erf(t * 0.7071067811865476))


def _dinv_of(dp_ref):
    deg = dp_ref[0][:, :1] + dp_ref[1][:, :1] + 1.0
    return lax.rsqrt(deg)


_ROW = lambda i: (i, 0)
_FIX = lambda i: (0, 0)
_P0 = lambda i: (0, i, 0)
_P1 = lambda i: (1, i, 0)


def _spec(shape, imap):
    return pl.BlockSpec(shape, imap)


def _pair_spec():
    return pl.BlockSpec((2, _BLK, _D), lambda i: (0, i, 0))


def _deg_spec():
    return pl.BlockSpec((2, _BLK, _D), lambda i: (0, i, 0))


def _tc0_body(x_ref, pw_ref, pb_ref, g1_ref, b1_ref, dp_ref, w0_ref,
              h0_ref, gs0_ref):
    h = _dot(x_ref[...], pw_ref[...]) + pb_ref[...]
    h = _ln(h, g1_ref[...], b1_ref[...])
    dinv = _dinv_of(dp_ref)
    h0_ref[...] = h
    gs0_ref[...] = _dot(h, w0_ref[...]) * dinv


def _tc_proj(x, proj_W, proj_b, ln1_g, ln1_b, degp, W0):
    return pl.pallas_call(
        _tc0_body,
        grid=(_GRID,),
        in_specs=[
            _spec((_BLK, _D), _ROW),
            _spec((_D, _D), _FIX),
            _spec((1, _D), _FIX),
            _spec((1, _D), _FIX),
            _spec((1, _D), _FIX),
            _deg_spec(),
            _spec((_D, _D), _FIX),
        ],
        out_specs=[_spec((_BLK, _D), _ROW)] * 2,
        out_shape=[jax.ShapeDtypeStruct((_N, _D), jnp.float32)] * 2,
    )(x, proj_W, proj_b, ln1_g, ln1_b, degp, W0)


def _hop_body(p_ref, gsp_ref, dp_ref, b_ref, wn_ref, h_ref, gs_ref):
    dinv = _dinv_of(dp_ref)
    agg = p_ref[0] + p_ref[1] + gsp_ref[...]
    h = _gelu(agg * dinv + b_ref[...])
    h_ref[...] = h
    gs_ref[...] = _dot(h, wn_ref[...]) * dinv


def _tc_hop(p, gsp, degp, bias, Wn):
    return pl.pallas_call(
        _hop_body,
        grid=(_GRID,),
        in_specs=[
            _pair_spec(),
            _spec((_BLK, _D), _ROW),
            _deg_spec(),
            _spec((1, _D), _FIX),
            _spec((_D, _D), _FIX),
        ],
        out_specs=[_spec((_BLK, _D), _ROW)] * 2,
        out_shape=[jax.ShapeDtypeStruct((_N, _D), jnp.float32)] * 2,
    )(p, gsp, degp, bias, Wn)


def _fin_body(p_ref, gs2_ref, dp_ref, b2_ref, h0_ref, h1_ref, h2_ref,
              w1_ref, b1m_ref, w2_ref, b2m_ref, g2_ref, bb2_ref, out_ref):
    dinv = _dinv_of(dp_ref)
    agg = p_ref[0] + p_ref[1] + gs2_ref[...]
    h3 = _gelu(agg * dinv + b2_ref[...])
    w1 = w1_ref[...]
    m = (_dot(h0_ref[...], w1[0:_D]) + _dot(h1_ref[...], w1[_D:2 * _D])
         + _dot(h2_ref[...], w1[2 * _D:3 * _D]) + _dot(h3, w1[3 * _D:4 * _D])
         + b1m_ref[...])
    m = _gelu(m)
    o = _dot(m, w2_ref[...]) + b2m_ref[...]
    out_ref[...] = _ln(o, g2_ref[...], bb2_ref[...])


def _tc_final(p, gs2, degp, b2, h0, h1, h2,
              mlp_W1, mlp_b1, mlp_W2, mlp_b2, ln2_g, ln2_b):
    return pl.pallas_call(
        _fin_body,
        grid=(_GRID,),
        in_specs=[
            _pair_spec(),
            _spec((_BLK, _D), _ROW),
            _deg_spec(),
            _spec((1, _D), _FIX),
            _spec((_BLK, _D), _ROW),
            _spec((_BLK, _D), _ROW),
            _spec((_BLK, _D), _ROW),
            _spec((4 * _D, _D), _FIX),
            _spec((1, _D), _FIX),
            _spec((_D, _D), _FIX),
            _spec((1, _D), _FIX),
            _spec((1, _D), _FIX),
            _spec((1, _D), _FIX),
        ],
        out_specs=_spec((_BLK, _D), _ROW),
        out_shape=jax.ShapeDtypeStruct((_N, _D), jnp.float32),
    )(p, gs2, degp, b2, h0, h1, h2,
      mlp_W1, mlp_b1, mlp_W2, mlp_b2, ln2_g, ln2_b)


def kernel(x, edge_index, proj_W, proj_b, ln1_g, ln1_b, gcn_Ws, gcn_bs,
           mlp_W1, mlp_b1, mlp_W2, mlp_b2, ln2_g, ln2_b):
    f32 = jnp.float32
    padw = _EPW - _E // _NW
    pad_src = jnp.zeros((_NW, padw), edge_index.dtype)
    pad_dst = jnp.broadcast_to(
        _N + jnp.arange(padw, dtype=edge_index.dtype), (_NW, padw))
    src1 = jnp.concatenate(
        [edge_index[0].reshape(_NW, _E // _NW), pad_src], axis=1).reshape(_EPAD)
    dst1 = jnp.concatenate(
        [edge_index[1].reshape(_NW, _E // _NW), pad_dst], axis=1).reshape(_EPAD)
    ones = jnp.ones((_K, _D), f32)
    zeros = jnp.zeros((_RPT, _D), f32)
    row = lambda v: v.reshape(1, _D)

    degp = _sc_scatter_rows(ones, src1, dst1, zeros, gather=False)

    h0, gs0 = _tc_proj(x, proj_W, row(proj_b), row(ln1_g), row(ln1_b),
                       degp, gcn_Ws[0])
    p = _sc_scatter_rows(gs0, src1, dst1, zeros, gather=True)
    h1, gs1 = _tc_hop(p, gs0, degp, row(gcn_bs[0]), gcn_Ws[1])
    p = _sc_scatter_rows(gs1, src1, dst1, zeros, gather=True)
    h2, gs2 = _tc_hop(p, gs1, degp, row(gcn_bs[1]), gcn_Ws[2])
    p = _sc_scatter_rows(gs2, src1, dst1, zeros, gather=True)
    return _tc_final(p, gs2, degp, row(gcn_bs[2]), h0, h1, h2,
                     mlp_W1, row(mlp_b1), mlp_W2, row(mlp_b2),
                     row(ln2_g), row(ln2_b))


# speedup vs baseline: 1.0094x; 1.0009x over previous
import functools

import jax
import jax.numpy as jnp
from jax import lax
from jax.experimental import pallas as pl
from jax.experimental.pallas import tpu as pltpu
from jax.experimental.pallas import tpu_sc as plsc

_N = 10000
_E = 320000
_D = 128
_DEPTH = 3
_K = 96
_C = 105
_NC = 2
_NS = 16
_NW = _NC * _NS
_EPW = _C * _K
_EPAD = _NW * _EPW
_NPAD = 10112
_RPT = _NPAD // _NS
_BLK = 2000
_GRID = _N // _BLK


def _sc_scatter_rows(table, src1, dst1, zeros, gather):
    w_dim = table.shape[1]
    mesh = plsc.VectorSubcoreMesh(core_axis_name="c", subcore_axis_name="s")

    @functools.partial(
        pl.kernel,
        out_type=jax.ShapeDtypeStruct((_NC, _NPAD, w_dim), jnp.float32),
        mesh=mesh,
        scratch_types=(
            [pltpu.VMEM((_EPW,), jnp.int32)]
            + [pltpu.VMEM((_K,), jnp.int32)] * _DEPTH
            + [pltpu.VMEM((_K, w_dim), jnp.float32)] * _DEPTH
            + [pltpu.SemaphoreType.DMA] * (2 * _DEPTH)
            + [pltpu.VMEM_SHARED((_NPAD, w_dim), jnp.float32)]
        ),
    )
    def k(table_hbm, src_hbm, dst_hbm, zeros_hbm, out_hbm, srcv, *rest):
        dbufs = rest[:_DEPTH]
        rbufs = rest[_DEPTH:2 * _DEPTH]
        gsems = rest[2 * _DEPTH:3 * _DEPTH]
        dsems = rest[3 * _DEPTH:4 * _DEPTH]
        accum = rest[-1]
        c = lax.axis_index("c")
        s = lax.axis_index("s")
        w = c * _NS + s
        base = w * _EPW

        pltpu.sync_copy(zeros_hbm, accum.at[pl.ds(s * _RPT, _RPT)])
        if gather:
            pltpu.sync_copy(src_hbm.at[pl.ds(base, _EPW)], srcv)
        else:
            pltpu.sync_copy(table_hbm, rbufs[0])

        def issue_d(j, b):
            pltpu.async_copy(dst_hbm.at[pl.ds(base + j * _K, _K)],
                             dbufs[b], dsems[b])

        def wait_d(j, b):
            pltpu.make_async_copy(dst_hbm.at[pl.ds(base + j * _K, _K)],
                                  dbufs[b], dsems[b]).wait()

        def issue_g(j, b):
            pltpu.async_copy(table_hbm.at[srcv.at[pl.ds(j * _K, _K)]],
                             rbufs[b], gsems[b])

        def wait_g(j, b):
            pltpu.make_async_copy(table_hbm.at[srcv.at[pl.ds(j * _K, _K)]],
                                  rbufs[b], gsems[b]).wait()

        for b in range(_DEPTH):
            issue_d(b, b)
            if gather:
                issue_g(b, b)
        plsc.subcore_barrier()

        if gather:
            def body(i, carry):
                j0 = _DEPTH * i
                for b in range(_DEPTH):
                    wait_g(j0 + b, b)
                    wait_d(j0 + b, b)
                    pltpu.sync_copy(rbufs[b], accum.at[dbufs[b]], add=True)
                    issue_g(j0 + b + _DEPTH, b)
                    issue_d(j0 + b + _DEPTH, b)
                return carry
        else:
            def body(i, carry):
                j0 = _DEPTH * i
                for b in range(_DEPTH):
                    wait_d(j0 + b, b)
                    pltpu.sync_copy(rbufs[0], accum.at[dbufs[b]], add=True)
                    issue_d(j0 + b + _DEPTH, b)
                return carry

        lax.fori_loop(0, _C // _DEPTH - 1, body, 0)
        for b in range(_DEPTH):
            j = _C - _DEPTH + b
            if gather:
                wait_g(j, b)
                wait_d(j, b)
                pltpu.sync_copy(rbufs[b], accum.at[dbufs[b]], add=True)
            else:
                wait_d(j, b)
                pltpu.sync_copy(rbufs[0], accum.at[dbufs[b]], add=True)
        plsc.subcore_barrier()
        pltpu.sync_copy(accum.at[pl.ds(s * _RPT, _RPT)],
                        out_hbm.at[c, pl.ds(s * _RPT, _RPT)])

    return k(table, src1, dst1, zeros)


def _dot(a, b):
    return lax.dot_general(a, b, (((1,), (0,)), ((), ())),
                           preferred_element_type=jnp.float32)


def _ln(t, g, b):
    m = jnp.mean(t, axis=-1, keepdims=True)
    v = jnp.mean((t - m) ** 2, axis=-1, keepdims=True)
    return (t - m) * lax.rsqrt(v + 1e-5) * g + b


def _gelu(t):
    return 0.5 * t * (1.0 + lax.erf(t * 0.7071067811865476))


def _dinv_of(dp_ref):
    deg = dp_ref[0][:, :1] + dp_ref[1][:, :1] + 1.0
    return lax.rsqrt(deg)


_ROW = lambda i: (i, 0)
_FIX = lambda i: (0, 0)


def _spec(shape, imap):
    return pl.BlockSpec(shape, imap)


def _pair_spec():
    return pl.BlockSpec((2, _BLK, _D), lambda i: (0, i, 0))


def _deg_spec():
    return pl.BlockSpec((2, _BLK, _D), lambda i: (0, i, 0))


def _tc0_body(x_ref, pw_ref, pb_ref, g1_ref, b1_ref, dp_ref, w0_ref,
              h0_ref, gs0_ref):
    h = _dot(x_ref[...], pw_ref[...]) + pb_ref[...]
    h = _ln(h, g1_ref[...], b1_ref[...])
    dinv = _dinv_of(dp_ref)
    h0_ref[...] = h
    gs0_ref[...] = _dot(h, w0_ref[...]) * dinv


def _tc_proj(x, proj_W, proj_b, ln1_g, ln1_b, degp, W0):
    return pl.pallas_call(
        _tc0_body,
        grid=(_GRID,),
        in_specs=[
            _spec((_BLK, _D), _ROW),
            _spec((_D, _D), _FIX),
            _spec((1, _D), _FIX),
            _spec((1, _D), _FIX),
            _spec((1, _D), _FIX),
            _deg_spec(),
            _spec((_D, _D), _FIX),
        ],
        out_specs=[_spec((_BLK, _D), _ROW)] * 2,
        out_shape=[jax.ShapeDtypeStruct((_N, _D), jnp.float32)] * 2,
    )(x, proj_W, proj_b, ln1_g, ln1_b, degp, W0)


def _hop_body(p_ref, gsp_ref, dp_ref, b_ref, wn_ref, h_ref, gs_ref):
    dinv = _dinv_of(dp_ref)
    agg = p_ref[0] + p_ref[1] + gsp_ref[...]
    h = _gelu(agg * dinv + b_ref[...])
    h_ref[...] = h
    gs_ref[...] = _dot(h, wn_ref[...]) * dinv


def _tc_hop(p, gsp, degp, bias, Wn):
    return pl.pallas_call(
        _hop_body,
        grid=(_GRID,),
        in_specs=[
            _pair_spec(),
            _spec((_BLK, _D), _ROW),
            _deg_spec(),
            _spec((1, _D), _FIX),
            _spec((_D, _D), _FIX),
        ],
        out_specs=[_spec((_BLK, _D), _ROW)] * 2,
        out_shape=[jax.ShapeDtypeStruct((_N, _D), jnp.float32)] * 2,
    )(p, gsp, degp, bias, Wn)


def _fin_body(p_ref, gs2_ref, dp_ref, b2_ref, h0_ref, h1_ref, h2_ref,
              w1_ref, b1m_ref, w2_ref, b2m_ref, g2_ref, bb2_ref, out_ref):
    dinv = _dinv_of(dp_ref)
    agg = p_ref[0] + p_ref[1] + gs2_ref[...]
    h3 = _gelu(agg * dinv + b2_ref[...])
    w1 = w1_ref[...]
    m = (_dot(h0_ref[...], w1[0:_D]) + _dot(h1_ref[...], w1[_D:2 * _D])
         + _dot(h2_ref[...], w1[2 * _D:3 * _D]) + _dot(h3, w1[3 * _D:4 * _D])
         + b1m_ref[...])
    m = _gelu(m)
    o = _dot(m, w2_ref[...]) + b2m_ref[...]
    out_ref[...] = _ln(o, g2_ref[...], bb2_ref[...])


def _tc_final(p, gs2, degp, b2, h0, h1, h2,
              mlp_W1, mlp_b1, mlp_W2, mlp_b2, ln2_g, ln2_b):
    return pl.pallas_call(
        _fin_body,
        grid=(_GRID,),
        in_specs=[
            _pair_spec(),
            _spec((_BLK, _D), _ROW),
            _deg_spec(),
            _spec((1, _D), _FIX),
            _spec((_BLK, _D), _ROW),
            _spec((_BLK, _D), _ROW),
            _spec((_BLK, _D), _ROW),
            _spec((4 * _D, _D), _FIX),
            _spec((1, _D), _FIX),
            _spec((_D, _D), _FIX),
            _spec((1, _D), _FIX),
            _spec((1, _D), _FIX),
            _spec((1, _D), _FIX),
        ],
        out_specs=_spec((_BLK, _D), _ROW),
        out_shape=jax.ShapeDtypeStruct((_N, _D), jnp.float32),
    )(p, gs2, degp, b2, h0, h1, h2,
      mlp_W1, mlp_b1, mlp_W2, mlp_b2, ln2_g, ln2_b)


def kernel(x, edge_index, proj_W, proj_b, ln1_g, ln1_b, gcn_Ws, gcn_bs,
           mlp_W1, mlp_b1, mlp_W2, mlp_b2, ln2_g, ln2_b):
    f32 = jnp.float32
    padw = _EPW - _E // _NW
    pad_src = jnp.zeros((_NW, padw), edge_index.dtype)
    pad_dst = jnp.broadcast_to(
        _N + jnp.arange(padw, dtype=edge_index.dtype), (_NW, padw))
    src1 = jnp.concatenate(
        [edge_index[0].reshape(_NW, _E // _NW), pad_src], axis=1).reshape(_EPAD)
    dst1 = jnp.concatenate(
        [edge_index[1].reshape(_NW, _E // _NW), pad_dst], axis=1).reshape(_EPAD)
    ones = jnp.ones((_K, _D), f32)
    zeros = jnp.zeros((_RPT, _D), f32)
    row = lambda v: v.reshape(1, _D)

    degp = _sc_scatter_rows(ones, src1, dst1, zeros, gather=False)

    h0, gs0 = _tc_proj(x, proj_W, row(proj_b), row(ln1_g), row(ln1_b),
                       degp, gcn_Ws[0])
    p = _sc_scatter_rows(gs0, src1, dst1, zeros, gather=True)
    h1, gs1 = _tc_hop(p, gs0, degp, row(gcn_bs[0]), gcn_Ws[1])
    p = _sc_scatter_rows(gs1, src1, dst1, zeros, gather=True)
    h2, gs2 = _tc_hop(p, gs1, degp, row(gcn_bs[1]), gcn_Ws[2])
    p = _sc_scatter_rows(gs2, src1, dst1, zeros, gather=True)
    return _tc_final(p, gs2, degp, row(gcn_bs[2]), h0, h1, h2,
                     mlp_W1, row(mlp_b1), mlp_W2, row(mlp_b2),
                     row(ln2_g), row(ln2_b))
